# baseline (speedup 1.0000x reference)
import jax
import jax.numpy as jnp
from jax import lax
from jax.experimental import pallas as pl
from jax.experimental.pallas import tpu as pltpu
from jax.experimental.pallas import tpu_sc as plsc

N_NODE = 10000
N_EDGE = 160000
D = 256
DH = D // 2
NC = 2
NS = 16
N_PAD = 10240
ROWS_PER_TILE = N_PAD // NS
EPT = N_EDGE // NS
EPT_P = 10240
FB = 128
N_CHUNK = EPT_P // FB
FBC = 80
CNT_PER_WORKER = 5120
CNT_TOTAL = NC * NS * CNT_PER_WORKER
N_CCHUNK = CNT_PER_WORKER // FBC
PAD_DST = 10016
BLK = 1280


def _sc_body(tbl_user, tbl_item, src_f, dst_f, cdst_f, src_c, dst_c, cdst_c,
             src_cb, dst_cb, cdst_cb, zfeat, ones_hbm,
             s_f, s_c, s_cb, c_f, c_c, c_cb,
             acc, rows0, onesb, sidx0, didx0, cidx, sem0):
    c = lax.axis_index("c")
    s = lax.axis_index("s")
    drain = pl.multiple_of(s * ROWS_PER_TILE, 8)
    out_off = pl.multiple_of(c * N_PAD + s * ROWS_PER_TILE, 8)

    pltpu.sync_copy(ones_hbm, onesb)

    for e_src, e_dst, e_cdst, tbl, s_out, c_out in (
        (src_f, dst_f, cdst_f, tbl_user, s_f, c_f),
        (src_c, dst_c, cdst_c, tbl_user, s_c, c_c),
        (src_cb, dst_cb, cdst_cb, tbl_item, s_cb, c_cb),
    ):
        pltpu.sync_copy(zfeat.at[pl.ds(drain, ROWS_PER_TILE)],
                        acc.at[pl.ds(drain, ROWS_PER_TILE)])
        plsc.subcore_barrier()

        @pl.loop(0, N_CHUNK)
        def fbody(j, e_src=e_src, e_dst=e_dst, tbl=tbl):
            fo = pl.multiple_of(c * (NS * EPT_P) + s * EPT_P + j * FB, 8)
            fd = pl.multiple_of(s * EPT_P + j * FB, 8)
            pltpu.sync_copy(e_src.at[pl.ds(fo, FB)], sidx0)
            pltpu.sync_copy(e_dst.at[pl.ds(fd, FB)], didx0)
            pltpu.async_copy(tbl.at[sidx0], rows0, sem0).wait()
            pltpu.sync_copy(rows0, acc.at[didx0], add=True)

        plsc.subcore_barrier()
        pltpu.sync_copy(acc.at[pl.ds(drain, ROWS_PER_TILE)],
                        s_out.at[pl.ds(out_off, ROWS_PER_TILE)])
        pltpu.sync_copy(zfeat.at[pl.ds(drain, ROWS_PER_TILE)],
                        acc.at[pl.ds(drain, ROWS_PER_TILE)])
        plsc.subcore_barrier()

        @pl.loop(0, N_CCHUNK)
        def cbody(j, e_cdst=e_cdst):
            co = pl.multiple_of((c * NS + s) * CNT_PER_WORKER + j * FBC, 8)
            pltpu.sync_copy(e_cdst.at[pl.ds(co, FBC)], cidx)
            pltpu.sync_copy(onesb, acc.at[cidx], add=True)

        plsc.subcore_barrier()
        pltpu.sync_copy(acc.at[pl.ds(drain, ROWS_PER_TILE)],
                        c_out.at[pl.ds(out_off, ROWS_PER_TILE)])


def _sc_agg(tbl_user, tbl_item, edge_rows, zfeat, ones):
    mesh = plsc.VectorSubcoreMesh(core_axis_name="c", subcore_axis_name="s",
                                  num_cores=NC, num_subcores=NS)
    f32 = jnp.float32
    call = pl.kernel(
        _sc_body,
        out_type=[jax.ShapeDtypeStruct((NC * N_PAD, DH), f32)] * 6,
        mesh=mesh,
        scratch_types=[
            pltpu.VMEM_SHARED((N_PAD, DH), f32),
            pltpu.VMEM((FB, DH), f32),
            pltpu.VMEM((FBC, DH), f32),
            pltpu.VMEM((FB,), jnp.int32),
            pltpu.VMEM((FB,), jnp.int32),
            pltpu.VMEM((FBC,), jnp.int32),
            pltpu.SemaphoreType.DMA,
        ],
    )
    return call(tbl_user, tbl_item, *edge_rows, zfeat, ones)


def _tc_body(sf0, sf1, cf0, cf1, scb0, scb1, ccb0, ccb1, sc0, sc1, cc0, cc1,
             Wf, bf, Wcb, bcb, Wc, bc, hu, hi):
    f32 = jnp.float32

    def rel(s0, s1, c0, c1, W, b):
        cnt = c0[:, :1] + c1[:, :1]
        r = 1.0 / jnp.maximum(cnt, 1.0)
        mask = (cnt > 0.0).astype(f32)
        x0 = s0[...] * r
        x1 = s1[...] * r
        Wm = W[...]
        h = lax.dot_general(x0, Wm[:, :DH], (((1,), (1,)), ((), ())),
                            preferred_element_type=f32)
        h += lax.dot_general(x1, Wm[:, DH:], (((1,), (1,)), ((), ())),
                             preferred_element_type=f32)
        return h + b[...] * mask

    hu[...] = (rel(sf0, sf1, cf0, cf1, Wf, bf)
               + rel(scb0, scb1, ccb0, ccb1, Wcb, bcb))
    hi[...] = rel(sc0, sc1, cc0, cc1, Wc, bc)


def _tc_epilogue(s_f, s_c, s_cb, c_f, c_c, c_cb,
                 W_f, b_f, W_c, b_c, W_cb, b_cb):
    f32 = jnp.float32
    half0 = pl.BlockSpec((BLK, DH), lambda i: (i, 0))
    half1 = pl.BlockSpec((BLK, DH), lambda i: (N_PAD // BLK + i, 0))
    wspec = pl.BlockSpec((D, D), lambda i: (0, 0))
    bspec = pl.BlockSpec((1, D), lambda i: (0, 0))
    ospec = pl.BlockSpec((BLK, D), lambda i: (i, 0))
    grid = (N_PAD // BLK,)
    out = pl.pallas_call(
        _tc_body,
        grid=grid,
        in_specs=[half0, half1] * 6 + [wspec, bspec] * 3,
        out_specs=[ospec, ospec],
        out_shape=[jax.ShapeDtypeStruct((N_NODE, D), f32)] * 2,
    )(s_f, s_f, c_f, c_f, s_cb, s_cb, c_cb, c_cb, s_c, s_c, c_c, c_c,
      W_f, b_f.reshape(1, D), W_cb, b_cb.reshape(1, D), W_c, b_c.reshape(1, D))
    return out


def kernel(feat_user, feat_item, edges_follows, edges_clicks,
           edges_clicked_by, W_follows, b_follows, W_clicks, b_clicks,
           W_clicked_by, b_clicked_by):
    f32 = jnp.float32
    i32 = jnp.int32
    tbl_u = feat_user.reshape(2 * N_NODE, DH)
    tbl_i = feat_item.reshape(2 * N_NODE, DH)
    zfeat = jnp.zeros((N_PAD, DH), f32)
    ones = jnp.ones((FBC, DH), f32)
    pad_cycle = N_NODE + jnp.arange(EPT_P - EPT, dtype=i32) % (N_PAD - N_NODE)
    cpad = N_NODE + jnp.arange(CNT_TOTAL - N_EDGE, dtype=i32) % (N_PAD - N_NODE)

    def pad_tiles(x, n_tiles, pad_vals):
        x = x.reshape(n_tiles, EPT)
        pad = jnp.broadcast_to(pad_vals, (n_tiles, EPT_P - EPT)).astype(i32)
        return jnp.concatenate([x, pad], axis=1).reshape(-1)

    edge_rows = []
    for e in (edges_follows, edges_clicks, edges_clicked_by):
        e = e.astype(i32)
        src2 = jnp.concatenate([e[0] * 2, e[0] * 2 + 1])
        edge_rows += [pad_tiles(src2, NC * NS, jnp.zeros((), i32)),
                      pad_tiles(e[1], NS, pad_cycle),
                      jnp.concatenate([e[1], cpad])]
    s_f, s_c, s_cb, c_f, c_c, c_cb = _sc_agg(tbl_u, tbl_i, edge_rows,
                                             zfeat, ones)
    return tuple(_tc_epilogue(s_f, s_c, s_cb, c_f, c_c, c_cb,
                              W_follows, b_follows, W_clicks, b_clicks,
                              W_clicked_by, b_clicked_by))


# speedup vs baseline: 2.1241x; 2.1241x over previous
import jax
import jax.numpy as jnp
from jax import lax
from jax.experimental import pallas as pl
from jax.experimental.pallas import tpu as pltpu
from jax.experimental.pallas import tpu_sc as plsc

N_NODE = 10000
N_EDGE = 160000
D = 256
DH = D // 2
NC = 2
NS = 16
N_PAD = 10240
ROWS_PER_TILE = N_PAD // NS
EPT = N_EDGE // NS
EPT_P = EPT
FB = 200
N_CHUNK = EPT_P // FB
FBC = 80
CNT_PER_WORKER = 5120
CNT_TOTAL = NC * NS * CNT_PER_WORKER
N_CCHUNK = CNT_PER_WORKER // FBC
PAD_DST = 10016
BLK = 1280


def _sc_body(tbl_user, tbl_item, src_f, dst_f, cdst_f, src_c, dst_c, cdst_c,
             src_cb, dst_cb, cdst_cb, zfeat, ones_hbm,
             s_f, s_c, s_cb, c_f, c_c, c_cb,
             acc, rows0, onesb, sidx0, didx0, cidx, sem0):
    c = lax.axis_index("c")
    s = lax.axis_index("s")
    drain = pl.multiple_of(s * ROWS_PER_TILE, 8)
    out_off = pl.multiple_of(c * N_PAD + s * ROWS_PER_TILE, 8)

    pltpu.sync_copy(ones_hbm, onesb)

    for e_src, e_dst, e_cdst, tbl, s_out, c_out in (
        (src_f, dst_f, cdst_f, tbl_user, s_f, c_f),
        (src_c, dst_c, cdst_c, tbl_user, s_c, c_c),
        (src_cb, dst_cb, cdst_cb, tbl_item, s_cb, c_cb),
    ):
        pltpu.sync_copy(zfeat.at[pl.ds(drain, ROWS_PER_TILE)],
                        acc.at[pl.ds(drain, ROWS_PER_TILE)])
        plsc.subcore_barrier()

        @pl.loop(0, N_CHUNK)
        def fbody(j, e_src=e_src, e_dst=e_dst, tbl=tbl):
            fo = pl.multiple_of(c * (NS * EPT_P) + s * EPT_P + j * FB, 8)
            fd = pl.multiple_of(s * EPT_P + j * FB, 8)
            pltpu.sync_copy(e_src.at[pl.ds(fo, FB)], sidx0)
            pltpu.sync_copy(e_dst.at[pl.ds(fd, FB)], didx0)
            pltpu.async_copy(tbl.at[sidx0], rows0, sem0).wait()
            pltpu.sync_copy(rows0, acc.at[didx0], add=True)

        plsc.subcore_barrier()
        pltpu.sync_copy(acc.at[pl.ds(drain, ROWS_PER_TILE)],
                        s_out.at[pl.ds(out_off, ROWS_PER_TILE)])
        pltpu.sync_copy(zfeat.at[pl.ds(drain, ROWS_PER_TILE)],
                        acc.at[pl.ds(drain, ROWS_PER_TILE)])
        plsc.subcore_barrier()

        @pl.loop(0, N_CCHUNK)
        def cbody(j, e_cdst=e_cdst):
            co = pl.multiple_of((c * NS + s) * CNT_PER_WORKER + j * FBC, 8)
            pltpu.sync_copy(e_cdst.at[pl.ds(co, FBC)], cidx)
            pltpu.sync_copy(onesb, acc.at[cidx], add=True)

        plsc.subcore_barrier()
        pltpu.sync_copy(acc.at[pl.ds(drain, ROWS_PER_TILE)],
                        c_out.at[pl.ds(out_off, ROWS_PER_TILE)])


def _sc_agg(tbl_user, tbl_item, edge_rows, zfeat, ones):
    mesh = plsc.VectorSubcoreMesh(core_axis_name="c", subcore_axis_name="s",
                                  num_cores=NC, num_subcores=NS)
    f32 = jnp.float32
    call = pl.kernel(
        _sc_body,
        out_type=[jax.ShapeDtypeStruct((NC * N_PAD, DH), f32)] * 6,
        mesh=mesh,
        scratch_types=[
            pltpu.VMEM_SHARED((N_PAD, DH), f32),
            pltpu.VMEM((FB, DH), f32),
            pltpu.VMEM((FBC, DH), f32),
            pltpu.VMEM((FB,), jnp.int32),
            pltpu.VMEM((FB,), jnp.int32),
            pltpu.VMEM((FBC,), jnp.int32),
            pltpu.SemaphoreType.DMA,
        ],
    )
    return call(tbl_user, tbl_item, *edge_rows, zfeat, ones)


def _tc_body(sf0, sf1, cf0, cf1, scb0, scb1, ccb0, ccb1, sc0, sc1, cc0, cc1,
             Wf, bf, Wcb, bcb, Wc, bc, hu, hi):
    f32 = jnp.float32

    def rel(s0, s1, c0, c1, W, b):
        cnt = c0[:, :1] + c1[:, :1]
        r = 1.0 / jnp.maximum(cnt, 1.0)
        mask = (cnt > 0.0).astype(f32)
        x0 = s0[...] * r
        x1 = s1[...] * r
        Wm = W[...]
        h = lax.dot_general(x0, Wm[:, :DH], (((1,), (1,)), ((), ())),
                            preferred_element_type=f32)
        h += lax.dot_general(x1, Wm[:, DH:], (((1,), (1,)), ((), ())),
                             preferred_element_type=f32)
        return h + b[...] * mask

    hu[...] = (rel(sf0, sf1, cf0, cf1, Wf, bf)
               + rel(scb0, scb1, ccb0, ccb1, Wcb, bcb))
    hi[...] = rel(sc0, sc1, cc0, cc1, Wc, bc)


def _tc_epilogue(s_f, s_c, s_cb, c_f, c_c, c_cb,
                 W_f, b_f, W_c, b_c, W_cb, b_cb):
    f32 = jnp.float32
    half0 = pl.BlockSpec((BLK, DH), lambda i: (i, 0))
    half1 = pl.BlockSpec((BLK, DH), lambda i: (N_PAD // BLK + i, 0))
    wspec = pl.BlockSpec((D, D), lambda i: (0, 0))
    bspec = pl.BlockSpec((1, D), lambda i: (0, 0))
    ospec = pl.BlockSpec((BLK, D), lambda i: (i, 0))
    grid = (N_PAD // BLK,)
    out = pl.pallas_call(
        _tc_body,
        grid=grid,
        in_specs=[half0, half1] * 6 + [wspec, bspec] * 3,
        out_specs=[ospec, ospec],
        out_shape=[jax.ShapeDtypeStruct((N_NODE, D), f32)] * 2,
    )(s_f, s_f, c_f, c_f, s_cb, s_cb, c_cb, c_cb, s_c, s_c, c_c, c_c,
      W_f, b_f.reshape(1, D), W_cb, b_cb.reshape(1, D), W_c, b_c.reshape(1, D))
    return out


def kernel(feat_user, feat_item, edges_follows, edges_clicks,
           edges_clicked_by, W_follows, b_follows, W_clicks, b_clicks,
           W_clicked_by, b_clicked_by):
    f32 = jnp.float32
    i32 = jnp.int32
    tbl_u = feat_user.reshape(2 * N_NODE, DH)
    tbl_i = feat_item.reshape(2 * N_NODE, DH)
    zfeat = jnp.zeros((N_PAD, DH), f32)
    ones = jnp.ones((FBC, DH), f32)
    pad_cycle = N_NODE + jnp.arange(EPT_P - EPT, dtype=i32) % (N_PAD - N_NODE)
    cpad = N_NODE + jnp.arange(CNT_TOTAL - N_EDGE, dtype=i32) % (N_PAD - N_NODE)

    def pad_tiles(x, n_tiles, pad_vals):
        x = x.reshape(n_tiles, EPT)
        pad = jnp.broadcast_to(pad_vals, (n_tiles, EPT_P - EPT)).astype(i32)
        return jnp.concatenate([x, pad], axis=1).reshape(-1)

    edge_rows = []
    for e in (edges_follows, edges_clicks, edges_clicked_by):
        e = e.astype(i32)
        src2 = jnp.concatenate([e[0] * 2, e[0] * 2 + 1])
        edge_rows += [pad_tiles(src2, NC * NS, jnp.zeros((), i32)),
                      pad_tiles(e[1], NS, pad_cycle),
                      jnp.concatenate([e[1], cpad])]
    s_f, s_c, s_cb, c_f, c_c, c_cb = _sc_agg(tbl_u, tbl_i, edge_rows,
                                             zfeat, ones)
    return tuple(_tc_epilogue(s_f, s_c, s_cb, c_f, c_c, c_cb,
                              W_follows, b_follows, W_clicks, b_clicks,
                              W_clicked_by, b_clicked_by))


# speedup vs baseline: 2.2318x; 1.0507x over previous
import jax
import jax.numpy as jnp
from jax import lax
from jax.experimental import pallas as pl
from jax.experimental.pallas import tpu as pltpu
from jax.experimental.pallas import tpu_sc as plsc

N_NODE = 10000
N_EDGE = 160000
D = 256
DH = D // 2
NC = 2
NS = 16
N_PAD = 10240
ROWS_PER_TILE = N_PAD // NS
EPT = N_EDGE // NS
EPT_P = EPT
FB = 200
N_CHUNK = EPT_P // FB
FBC = 160
CNT_PER_WORKER = 5120
CNT_TOTAL = NC * NS * CNT_PER_WORKER
N_CCHUNK = CNT_PER_WORKER // FBC
PAD_DST = 10016
BLK = 1280


def _sc_body(tbl_user, tbl_item, src_f, dst_f, cdst_f, src_c, dst_c, cdst_c,
             src_cb, dst_cb, cdst_cb, zfeat, ones_hbm,
             s_f, s_c, s_cb, c_f, c_c, c_cb,
             acc, rows0, onesb, sidx0, didx0, cidx, sem0):
    c = lax.axis_index("c")
    s = lax.axis_index("s")
    drain = pl.multiple_of(s * ROWS_PER_TILE, 8)
    out_off = pl.multiple_of(c * N_PAD + s * ROWS_PER_TILE, 8)

    pltpu.sync_copy(ones_hbm, onesb)

    for e_src, e_dst, e_cdst, tbl, s_out, c_out in (
        (src_f, dst_f, cdst_f, tbl_user, s_f, c_f),
        (src_c, dst_c, cdst_c, tbl_user, s_c, c_c),
        (src_cb, dst_cb, cdst_cb, tbl_item, s_cb, c_cb),
    ):
        pltpu.sync_copy(zfeat.at[pl.ds(drain, ROWS_PER_TILE)],
                        acc.at[pl.ds(drain, ROWS_PER_TILE)])
        plsc.subcore_barrier()

        @pl.loop(0, N_CHUNK)
        def fbody(j, e_src=e_src, e_dst=e_dst, tbl=tbl):
            fo = pl.multiple_of(c * (NS * EPT_P) + s * EPT_P + j * FB, 8)
            fd = pl.multiple_of(s * EPT_P + j * FB, 8)
            pltpu.sync_copy(e_src.at[pl.ds(fo, FB)], sidx0)
            pltpu.sync_copy(e_dst.at[pl.ds(fd, FB)], didx0)
            pltpu.async_copy(tbl.at[sidx0], rows0, sem0).wait()
            pltpu.sync_copy(rows0, acc.at[didx0], add=True)

        plsc.subcore_barrier()
        pltpu.sync_copy(acc.at[pl.ds(drain, ROWS_PER_TILE)],
                        s_out.at[pl.ds(out_off, ROWS_PER_TILE)])
        pltpu.sync_copy(zfeat.at[pl.ds(drain, ROWS_PER_TILE)],
                        acc.at[pl.ds(drain, ROWS_PER_TILE)])
        plsc.subcore_barrier()

        @pl.loop(0, N_CCHUNK)
        def cbody(j, e_cdst=e_cdst):
            co = pl.multiple_of((c * NS + s) * CNT_PER_WORKER + j * FBC, 8)
            pltpu.sync_copy(e_cdst.at[pl.ds(co, FBC)], cidx)
            pltpu.sync_copy(onesb, acc.at[cidx], add=True)

        plsc.subcore_barrier()
        pltpu.sync_copy(acc.at[pl.ds(drain, ROWS_PER_TILE)],
                        c_out.at[pl.ds(out_off, ROWS_PER_TILE)])


def _sc_agg(tbl_user, tbl_item, edge_rows, zfeat, ones):
    mesh = plsc.VectorSubcoreMesh(core_axis_name="c", subcore_axis_name="s",
                                  num_cores=NC, num_subcores=NS)
    f32 = jnp.float32
    call = pl.kernel(
        _sc_body,
        out_type=[jax.ShapeDtypeStruct((NC * N_PAD, DH), f32)] * 6,
        mesh=mesh,
        scratch_types=[
            pltpu.VMEM_SHARED((N_PAD, DH), f32),
            pltpu.VMEM((FB, DH), f32),
            pltpu.VMEM((FBC, DH), f32),
            pltpu.VMEM((FB,), jnp.int32),
            pltpu.VMEM((FB,), jnp.int32),
            pltpu.VMEM((FBC,), jnp.int32),
            pltpu.SemaphoreType.DMA,
        ],
    )
    return call(tbl_user, tbl_item, *edge_rows, zfeat, ones)


def _tc_body(sf0, sf1, cf0, cf1, scb0, scb1, ccb0, ccb1, sc0, sc1, cc0, cc1,
             Wf, bf, Wcb, bcb, Wc, bc, hu, hi):
    f32 = jnp.float32

    def rel(s0, s1, c0, c1, W, b):
        cnt = c0[:, :1] + c1[:, :1]
        r = 1.0 / jnp.maximum(cnt, 1.0)
        mask = (cnt > 0.0).astype(f32)
        x0 = s0[...] * r
        x1 = s1[...] * r
        Wm = W[...]
        h = lax.dot_general(x0, Wm[:, :DH], (((1,), (1,)), ((), ())),
                            preferred_element_type=f32)
        h += lax.dot_general(x1, Wm[:, DH:], (((1,), (1,)), ((), ())),
                             preferred_element_type=f32)
        return h + b[...] * mask

    hu[...] = (rel(sf0, sf1, cf0, cf1, Wf, bf)
               + rel(scb0, scb1, ccb0, ccb1, Wcb, bcb))
    hi[...] = rel(sc0, sc1, cc0, cc1, Wc, bc)


def _tc_epilogue(s_f, s_c, s_cb, c_f, c_c, c_cb,
                 W_f, b_f, W_c, b_c, W_cb, b_cb):
    f32 = jnp.float32
    half0 = pl.BlockSpec((BLK, DH), lambda i: (i, 0))
    half1 = pl.BlockSpec((BLK, DH), lambda i: (N_PAD // BLK + i, 0))
    wspec = pl.BlockSpec((D, D), lambda i: (0, 0))
    bspec = pl.BlockSpec((1, D), lambda i: (0, 0))
    ospec = pl.BlockSpec((BLK, D), lambda i: (i, 0))
    grid = (N_PAD // BLK,)
    out = pl.pallas_call(
        _tc_body,
        grid=grid,
        in_specs=[half0, half1] * 6 + [wspec, bspec] * 3,
        out_specs=[ospec, ospec],
        out_shape=[jax.ShapeDtypeStruct((N_NODE, D), f32)] * 2,
    )(s_f, s_f, c_f, c_f, s_cb, s_cb, c_cb, c_cb, s_c, s_c, c_c, c_c,
      W_f, b_f.reshape(1, D), W_cb, b_cb.reshape(1, D), W_c, b_c.reshape(1, D))
    return out


def kernel(feat_user, feat_item, edges_follows, edges_clicks,
           edges_clicked_by, W_follows, b_follows, W_clicks, b_clicks,
           W_clicked_by, b_clicked_by):
    f32 = jnp.float32
    i32 = jnp.int32
    tbl_u = feat_user.reshape(2 * N_NODE, DH)
    tbl_i = feat_item.reshape(2 * N_NODE, DH)
    zfeat = jnp.zeros((N_PAD, DH), f32)
    ones = jnp.ones((FBC, DH), f32)
    pad_cycle = N_NODE + jnp.arange(EPT_P - EPT, dtype=i32) % (N_PAD - N_NODE)
    cpad = N_NODE + jnp.arange(CNT_TOTAL - N_EDGE, dtype=i32) % (N_PAD - N_NODE)

    def pad_tiles(x, n_tiles, pad_vals):
        x = x.reshape(n_tiles, EPT)
        pad = jnp.broadcast_to(pad_vals, (n_tiles, EPT_P - EPT)).astype(i32)
        return jnp.concatenate([x, pad], axis=1).reshape(-1)

    edge_rows = []
    for e in (edges_follows, edges_clicks, edges_clicked_by):
        e = e.astype(i32)
        src2 = jnp.concatenate([e[0] * 2, e[0] * 2 + 1])
        edge_rows += [pad_tiles(src2, NC * NS, jnp.zeros((), i32)),
                      pad_tiles(e[1], NS, pad_cycle),
                      jnp.concatenate([e[1], cpad])]
    s_f, s_c, s_cb, c_f, c_c, c_cb = _sc_agg(tbl_u, tbl_i, edge_rows,
                                             zfeat, ones)
    return tuple(_tc_epilogue(s_f, s_c, s_cb, c_f, c_c, c_cb,
                              W_follows, b_follows, W_clicks, b_clicks,
                              W_clicked_by, b_clicked_by))
